# baseline (speedup 1.0000x reference)
import functools

import jax
import jax.numpy as jnp
from jax import lax
from jax.experimental import pallas as pl
from jax.experimental.pallas import tpu as pltpu
from jax.experimental.pallas import tpu_sc as plsc

BATCH = 16384
EMBED_DIM = 64

_NC = 2
_NS = 16
_NW = _NC * _NS
_B_PER_W = BATCH // _NW
_CHUNK = 128
_NCHUNK = _B_PER_W // _CHUNK

_mesh = plsc.VectorSubcoreMesh(core_axis_name="c", subcore_axis_name="s")


@functools.partial(
    pl.kernel,
    mesh=_mesh,
    out_type=(
        jax.ShapeDtypeStruct((BATCH, EMBED_DIM), jnp.float32),
        jax.ShapeDtypeStruct((BATCH, EMBED_DIM), jnp.float32),
    ),
    scratch_types=[
        pltpu.VMEM((_NCHUNK, _CHUNK), jnp.int32),
        pltpu.VMEM((_NCHUNK, _CHUNK), jnp.int32),
        pltpu.VMEM((_B_PER_W, EMBED_DIM), jnp.float32),
        pltpu.VMEM((_B_PER_W, EMBED_DIM), jnp.float32),
        pltpu.SemaphoreType.DMA,
    ],
    compiler_params=pltpu.CompilerParams(use_tc_tiling_on_sc=False),
)
def _two_tower_gather(uid_hbm, iid_hbm, utab_hbm, itab_hbm,
                      uout_hbm, iout_hbm,
                      uidx_v, iidx_v, urows_v, irows_v, sem):
    wid = lax.axis_index("s") * _NC + lax.axis_index("c")
    base = wid * _B_PER_W

    pltpu.sync_copy(uid_hbm.at[wid], uidx_v)
    pltpu.sync_copy(iid_hbm.at[wid], iidx_v)

    copies = []
    for j in range(_NCHUNK):
        dst = urows_v.at[pl.ds(j * _CHUNK, _CHUNK)]
        copies.append(pltpu.async_copy(utab_hbm.at[uidx_v.at[j]], dst, sem))
    for j in range(_NCHUNK):
        dst = irows_v.at[pl.ds(j * _CHUNK, _CHUNK)]
        copies.append(pltpu.async_copy(itab_hbm.at[iidx_v.at[j]], dst, sem))
    for c in copies:
        c.wait()

    pltpu.sync_copy(urows_v, uout_hbm.at[pl.ds(base, _B_PER_W)])
    pltpu.sync_copy(irows_v, iout_hbm.at[pl.ds(base, _B_PER_W)])


def kernel(user_ids, pos_item_ids, user_table, item_table):
    u3 = user_ids.astype(jnp.int32).reshape(_NW, _NCHUNK, _CHUNK)
    i3 = pos_item_ids.astype(jnp.int32).reshape(_NW, _NCHUNK, _CHUNK)
    return _two_tower_gather(u3, i3, user_table, item_table)


# speedup vs baseline: 1.5475x; 1.5475x over previous
import functools

import jax
import jax.numpy as jnp
from jax import lax
from jax.experimental import pallas as pl
from jax.experimental.pallas import tpu as pltpu
from jax.experimental.pallas import tpu_sc as plsc

BATCH = 16384
EMBED_DIM = 64

_NC = 2
_NS = 16
_NW = _NC * _NS
_W = BATCH // _NW
_K = 32
_NB = _W // _K

_mesh = plsc.VectorSubcoreMesh(core_axis_name="c", subcore_axis_name="s")


@functools.partial(
    pl.kernel,
    mesh=_mesh,
    out_type=(
        jax.ShapeDtypeStruct((BATCH, EMBED_DIM), jnp.float32),
        jax.ShapeDtypeStruct((BATCH, EMBED_DIM), jnp.float32),
    ),
    scratch_types=[
        pltpu.VMEM((_W,), jnp.int32),
        pltpu.VMEM((_W,), jnp.int32),
        pltpu.VMEM((2, _K, EMBED_DIM), jnp.float32),
        pltpu.SemaphoreType.DMA,
        pltpu.SemaphoreType.DMA,
    ],
)
def _two_tower_gather(uid_hbm, iid_hbm, utab_hbm, itab_hbm,
                      uout_hbm, iout_hbm,
                      uids_s, iids_s, rows_v, gsem, wsem):
    wid = lax.axis_index("s") * _NC + lax.axis_index("c")
    base = wid * _W

    pltpu.sync_copy(uid_hbm.at[wid], uids_s)
    pltpu.sync_copy(iid_hbm.at[wid], iids_s)

    for ids_s, tab, out_hbm in (
        (uids_s, utab_hbm, uout_hbm),
        (iids_s, itab_hbm, iout_hbm),
    ):
        def fire(g, b):
            for k in range(_K // 16):
                v16 = ids_s[pl.ds(g * _K + k * 16, 16)]
                for l in range(16):
                    pltpu.async_copy(
                        tab.at[v16[l]], rows_v.at[b, k * 16 + l], gsem)

        def drain(b):
            for j in range(_K):
                pltpu.make_async_copy(tab.at[0], rows_v.at[b, j], gsem).wait()

        fire(0, 0)

        @pl.loop(0, _NB)
        def _block(g):
            b = g % 2

            @pl.when(g >= 1)
            def _():
                pltpu.make_async_copy(
                    rows_v.at[1 - b], out_hbm.at[pl.ds(0, _K)], wsem).wait()

            @pl.when(g + 1 < _NB)
            def _():
                fire(g + 1, 1 - b)

            drain(b)
            pltpu.async_copy(
                rows_v.at[b], out_hbm.at[pl.ds(base + g * _K, _K)], wsem)

        pltpu.make_async_copy(
            rows_v.at[0], out_hbm.at[pl.ds(0, _K)], wsem).wait()


def kernel(user_ids, pos_item_ids, user_table, item_table):
    u2 = user_ids.astype(jnp.int32).reshape(_NW, _W)
    i2 = pos_item_ids.astype(jnp.int32).reshape(_NW, _W)
    return _two_tower_gather(u2, i2, user_table, item_table)
